# baseline (speedup 1.0000x reference)
import jax
import jax.numpy as jnp
from jax.experimental import pallas as pl
from jax.experimental.pallas import tpu as pltpu


def _body(x_ref, a0_ref, a1_ref, w_ref, o_ref, sc_ref):
    dn = (((1,), (0,)), ((), ()))
    f32 = jnp.float32
    bf16 = jnp.bfloat16
    N, wt = x_ref.shape
    gw, wc = w_ref.shape
    n_sub = wt // wc
    n_groups = gw // wc

    def stage(g, v):
        for k in range(n_sub):
            sc_ref[:, k * gw + g * wc:k * gw + (g + 1) * wc] = (
                v[:, k * wc:(k + 1) * wc])

    def mm(a_ref, v):
        outs = []
        for r in range(N // 256):
            p = jax.lax.dot_general(a_ref[r * 256:(r + 1) * 256, :], v, dn,
                                    preferred_element_type=f32)
            outs.append(p.astype(bf16))
        return jnp.concatenate(outs, axis=0)

    xb = x_ref[...]
    stage(0, xb)
    h1_0 = mm(a0_ref, xb)
    h1_1 = mm(a1_ref, xb)
    stage(1, h1_0)
    stage(3, h1_1)
    h2_0 = mm(a0_ref, h1_0)
    h2_1 = mm(a1_ref, h1_1)
    stage(2, h2_0)
    stage(4, h2_1)

    for k in range(n_sub):
        fs = sc_ref[:, k * gw:(k + 1) * gw]
        o_ref[:, k * wc:(k + 1) * wc] = jax.lax.dot_general(
            fs, w_ref[...], dn, preferred_element_type=f32).astype(bf16)


def kernel(x, adjs, W, b):
    B, C, N, T = x.shape
    out_ch, in_ch = W.shape
    n_groups = in_ch // C

    nbt = 4
    wc = nbt * C
    wt = 2 * wc
    cols = B * T * C
    grid = cols // wt

    bf16 = jnp.bfloat16
    xt = jnp.transpose(x, (2, 0, 3, 1)).reshape(N, cols).astype(bf16)
    a0 = adjs[0].T.astype(bf16)
    a1 = adjs[1].T.astype(bf16)
    eye = jnp.eye(nbt, dtype=W.dtype)
    wbd = jnp.concatenate(
        [jnp.kron(eye, W[:, g * C:(g + 1) * C].T) for g in range(n_groups)],
        axis=0).astype(bf16)

    out2d = pl.pallas_call(
        _body,
        grid=(grid,),
        in_specs=[
            pl.BlockSpec((N, wt), lambda j: (0, j)),
            pl.BlockSpec((N, N), lambda j: (0, 0)),
            pl.BlockSpec((N, N), lambda j: (0, 0)),
            pl.BlockSpec(wbd.shape, lambda j: (0, 0)),
        ],
        out_specs=pl.BlockSpec((N, wt), lambda j: (0, j)),
        out_shape=jax.ShapeDtypeStruct((N, cols), bf16),
        scratch_shapes=[pltpu.VMEM((N, 2 * n_groups * wc), bf16)],
    )(xt, a0, a1, wbd)

    out = out2d.reshape(N, B, T, out_ch).transpose(1, 3, 0, 2)
    return out.astype(jnp.float32) + b[None, :, None, None]
